# baseline (speedup 1.0000x reference)
import functools

import jax
import jax.numpy as jnp
from jax import lax
from jax.experimental import pallas as pl
from jax.experimental.pallas import tpu as pltpu
from jax.experimental.pallas import tpu_sc as plsc

NC = 2
NS = 16
L = 16
NW = NC * NS

NBINS = 4096
COLS = 2048
CROWS = 8
SAMPLE_FRAC = 16


def _scan_loop(x_hbm, buf0, buf1, sem0, sem1, rowbase, nchunks, process):
    pltpu.async_copy(x_hbm.at[pl.ds(rowbase, CROWS)], buf0, sem0)
    pltpu.async_copy(x_hbm.at[pl.ds(rowbase + CROWS, CROWS)], buf1, sem1)

    def _outer(g, c):
        row = rowbase + 2 * g * CROWS
        pltpu.make_async_copy(
            x_hbm.at[pl.ds(row, CROWS)], buf0, sem0).wait()
        process(buf0)

        @pl.when(2 * g + 2 < nchunks)
        def _():
            pltpu.async_copy(
                x_hbm.at[pl.ds(row + 2 * CROWS, CROWS)], buf0, sem0)

        pltpu.make_async_copy(
            x_hbm.at[pl.ds(row + CROWS, CROWS)], buf1, sem1).wait()
        process(buf1)

        @pl.when(2 * g + 3 < nchunks)
        def _():
            pltpu.async_copy(
                x_hbm.at[pl.ds(row + 3 * CROWS, CROWS)], buf1, sem1)
        return c
    lax.fori_loop(0, nchunks // 2, _outer, 0)


def _finish_hist(hist, outbuf, hist_hbm, wid):
    @plsc.parallel_loop(0, NBINS, step=L)
    def _red(j):
        acc = hist[pl.ds(j, L)]
        for lane in range(1, L):
            acc = acc + hist[pl.ds(lane * NBINS + j, L)]
        outbuf[pl.ds(j, L)] = acc

    pltpu.sync_copy(outbuf, hist_hbm.at[pl.ds(wid * NBINS, NBINS)])


def _zero_hist(hist):
    @plsc.parallel_loop(0, L * NBINS, step=L, unroll=8)
    def _zero(i):
        hist[pl.ds(i, L)] = jnp.zeros((L,), jnp.int32)


def _load_vreg(buf, i):
    return buf[lax.shift_right_logical(i, COLS.bit_length() - 1),
               pl.ds(lax.bitwise_and(i, COLS - 1), L)]


def _radix_body(shift_bin, shift_prefix, frac, x_hbm, prefix_hbm, hist_hbm,
                buf0, buf1, hist, outbuf, prefix_v, sem0, sem1):
    rows_per_w = x_hbm.shape[0] // NW
    nchunks = rows_per_w // CROWS // frac
    wid = lax.axis_index("s") * NC + lax.axis_index("c")
    rowbase = wid * rows_per_w

    laneoff = lax.iota(jnp.int32, L) * jnp.int32(NBINS)
    ones = jnp.ones((L,), jnp.int32)

    _zero_hist(hist)
    pltpu.sync_copy(prefix_hbm, prefix_v)
    pvec = prefix_v[...]

    def _process(buf):
        @plsc.parallel_loop(0, CROWS * COLS, step=L, unroll=8)
        def _inner(i):
            key = lax.bitcast_convert_type(lax.abs(_load_vreg(buf, i)),
                                           jnp.int32)
            t = lax.shift_right_logical(key, jnp.int32(shift_bin))
            idx = laneoff + lax.bitwise_and(t, jnp.int32(NBINS - 1))
            if shift_prefix >= 31:
                plsc.addupdate_scatter(hist, [idx], ones)
            else:
                m = lax.shift_right_logical(
                    t, jnp.int32(shift_prefix - shift_bin)) == pvec
                plsc.addupdate_scatter(hist, [idx], ones, mask=m)

    _scan_loop(x_hbm, buf0, buf1, sem0, sem1, rowbase, nchunks, _process)
    _finish_hist(hist, outbuf, hist_hbm, wid)


def _window_body(x_hbm, base_hbm, hist_hbm,
                 buf0, buf1, hist, outbuf, base_v, sem0, sem1):
    rows_per_w = x_hbm.shape[0] // NW
    nchunks = rows_per_w // CROWS
    wid = lax.axis_index("s") * NC + lax.axis_index("c")
    rowbase = wid * rows_per_w

    laneoff = lax.iota(jnp.int32, L) * jnp.int32(NBINS)
    ones = jnp.ones((L,), jnp.int32)

    _zero_hist(hist)
    pltpu.sync_copy(base_hbm, base_v)
    bvec = base_v[...]

    def _process(buf):
        @plsc.parallel_loop(0, CROWS * COLS, step=L, unroll=8)
        def _inner(i):
            key = lax.bitcast_convert_type(lax.abs(_load_vreg(buf, i)),
                                           jnp.int32)
            m = key >= bvec
            binv = lax.min(
                lax.shift_right_logical(key - bvec, jnp.int32(7)),
                jnp.int32(NBINS - 1))
            plsc.addupdate_scatter(hist, [laneoff + binv], ones, mask=m)

    _scan_loop(x_hbm, buf0, buf1, sem0, sem1, rowbase, nchunks, _process)
    _finish_hist(hist, outbuf, hist_hbm, wid)


def _sc_kernel(body):
    mesh = plsc.VectorSubcoreMesh(core_axis_name="c", subcore_axis_name="s",
                                  num_cores=NC, num_subcores=NS)
    return pl.kernel(
        body,
        out_type=jax.ShapeDtypeStruct((NW * NBINS,), jnp.int32),
        mesh=mesh,
        compiler_params=pltpu.CompilerParams(
            needs_layout_passes=False, use_tc_tiling_on_sc=True),
        scratch_types=[
            pltpu.VMEM((CROWS, COLS), jnp.float32),
            pltpu.VMEM((CROWS, COLS), jnp.float32),
            pltpu.VMEM((L * NBINS,), jnp.int32),
            pltpu.VMEM((NBINS,), jnp.int32),
            pltpu.VMEM((L,), jnp.int32),
            pltpu.SemaphoreType.DMA,
            pltpu.SemaphoreType.DMA,
        ],
    )


def _make_radix_kernel(shift_bin, shift_prefix, frac=1):
    return _sc_kernel(
        functools.partial(_radix_body, shift_bin, shift_prefix, frac))


def _select(hist, r):
    suffix = jnp.cumsum(hist[::-1])[::-1]
    b = jnp.sum(suffix >= r).astype(jnp.int32) - 1
    r_next = r - (suffix[b] - hist[b])
    return b, r_next


def _mask_body(kt_ref, x_ref, o_ref):
    bits = lax.bitcast_convert_type(x_ref[...], jnp.int32)
    key = jnp.bitwise_and(bits, jnp.int32(0x7FFFFFFF))
    o_ref[...] = jnp.where(key >= kt_ref[0], x_ref[...], jnp.float32(0.0))


def kernel(x):
    n = x.size
    k = max(1, int(n * (1.0 - 0.7)))
    rows = n // COLS
    x2d = x.reshape(rows, COLS)
    zeros16 = jnp.zeros((L,), jnp.int32)

    hs = _make_radix_kernel(19, 31, SAMPLE_FRAC)(x2d, zeros16)
    ks = max(1, k // SAMPLE_FRAC)
    bs, rs = _select(hs.reshape(NW, NBINS).sum(axis=0), jnp.int32(ks))
    cs = jnp.maximum(hs.reshape(NW, NBINS).sum(axis=0)[bs], 1)
    off = ((cs - rs).astype(jnp.float32) / cs.astype(jnp.float32)
           * jnp.float32(1 << 19)).astype(jnp.int32)
    t_est = (bs << 19) + jnp.clip(off, 0, (1 << 19) - 1)
    base = jnp.clip(t_est - (1 << 18), 0, 0x7FFFFFFF - (NBINS << 7))

    hw = _sc_kernel(_window_body)(x2d, jnp.full((L,), base, jnp.int32))
    suffix = jnp.cumsum(hw.reshape(NW, NBINS).sum(axis=0)[::-1])[::-1]
    j = jnp.sum(suffix >= k).astype(jnp.int32) - 1
    key_t_fast = base + (j << 7)
    missed = (j < 0) | (j >= NBINS - 1)

    def _exact(_):
        h1 = _make_radix_kernel(19, 31)(x2d, zeros16)
        b1, r1 = _select(h1.reshape(NW, NBINS).sum(axis=0), jnp.int32(k))
        h2 = _make_radix_kernel(7, 19)(x2d, jnp.full((L,), b1, jnp.int32))
        b2, _ = _select(h2.reshape(NW, NBINS).sum(axis=0), r1)
        return ((b1 << 12) | b2) << 7

    key_t = lax.cond(missed, _exact, lambda _: key_t_fast, None)

    blk = 512
    out = pl.pallas_call(
        _mask_body,
        grid=(rows // blk,),
        in_specs=[
            pl.BlockSpec(memory_space=pltpu.SMEM),
            pl.BlockSpec((blk, COLS), lambda i: (i, 0)),
        ],
        out_specs=pl.BlockSpec((blk, COLS), lambda i: (i, 0)),
        out_shape=jax.ShapeDtypeStruct((rows, COLS), jnp.float32),
    )(key_t.reshape(1), x2d)
    return out.reshape(x.shape)


# speedup vs baseline: 1.4660x; 1.4660x over previous
import functools

import jax
import jax.numpy as jnp
from jax import lax
from jax.experimental import pallas as pl
from jax.experimental.pallas import tpu as pltpu
from jax.experimental.pallas import tpu_sc as plsc

NC = 2
NS = 16
L = 16
NW = NC * NS

NBINS = 4096
COLS = 2048
CROWS = 8
SAMPLE_FRAC = 16
WBINS = NBINS - 8


def _scan_loop(x_hbm, buf0, buf1, sem0, sem1, rowbase, nchunks, process):
    pltpu.async_copy(x_hbm.at[pl.ds(rowbase, CROWS)], buf0, sem0)
    pltpu.async_copy(x_hbm.at[pl.ds(rowbase + CROWS, CROWS)], buf1, sem1)

    def _outer(g, c):
        row = rowbase + 2 * g * CROWS
        pltpu.make_async_copy(
            x_hbm.at[pl.ds(row, CROWS)], buf0, sem0).wait()
        process(buf0)

        @pl.when(2 * g + 2 < nchunks)
        def _():
            pltpu.async_copy(
                x_hbm.at[pl.ds(row + 2 * CROWS, CROWS)], buf0, sem0)

        pltpu.make_async_copy(
            x_hbm.at[pl.ds(row + CROWS, CROWS)], buf1, sem1).wait()
        process(buf1)

        @pl.when(2 * g + 3 < nchunks)
        def _():
            pltpu.async_copy(
                x_hbm.at[pl.ds(row + 3 * CROWS, CROWS)], buf1, sem1)
        return c
    lax.fori_loop(0, nchunks // 2, _outer, 0)


def _finish_hist(hist, outbuf, hist_hbm, wid):
    @plsc.parallel_loop(0, NBINS, step=L)
    def _red(j):
        acc = hist[pl.ds(j, L)]
        for lane in range(1, L):
            acc = acc + hist[pl.ds(lane * NBINS + j, L)]
        outbuf[pl.ds(j, L)] = acc

    pltpu.sync_copy(outbuf, hist_hbm.at[pl.ds(wid * NBINS, NBINS)])


def _zero_hist(hist):
    @plsc.parallel_loop(0, L * NBINS, step=L, unroll=8)
    def _zero(i):
        hist[pl.ds(i, L)] = jnp.zeros((L,), jnp.int32)


def _load_vreg(buf, i):
    return buf[lax.shift_right_logical(i, COLS.bit_length() - 1),
               pl.ds(lax.bitwise_and(i, COLS - 1), L)]


def _radix_body(shift_bin, shift_prefix, frac, x_hbm, prefix_hbm, hist_hbm,
                buf0, buf1, hist, outbuf, prefix_v, sem0, sem1):
    rows_per_w = x_hbm.shape[0] // NW
    nchunks = rows_per_w // CROWS // frac
    wid = lax.axis_index("s") * NC + lax.axis_index("c")
    rowbase = wid * rows_per_w

    laneoff = lax.iota(jnp.int32, L) * jnp.int32(NBINS)
    ones = jnp.ones((L,), jnp.int32)

    _zero_hist(hist)
    pltpu.sync_copy(prefix_hbm, prefix_v)
    pvec = prefix_v[...]

    def _process(buf):
        @plsc.parallel_loop(0, CROWS * COLS, step=L, unroll=8)
        def _inner(i):
            key = lax.bitcast_convert_type(lax.abs(_load_vreg(buf, i)),
                                           jnp.int32)
            t = lax.shift_right_logical(key, jnp.int32(shift_bin))
            idx = laneoff + lax.bitwise_and(t, jnp.int32(NBINS - 1))
            if shift_prefix >= 31:
                plsc.addupdate_scatter(hist, [idx], ones)
            else:
                m = lax.shift_right_logical(
                    t, jnp.int32(shift_prefix - shift_bin)) == pvec
                plsc.addupdate_scatter(hist, [idx], ones, mask=m)

    _scan_loop(x_hbm, buf0, buf1, sem0, sem1, rowbase, nchunks, _process)
    _finish_hist(hist, outbuf, hist_hbm, wid)


def _window_body(x_hbm, base_hbm, hist_hbm,
                 buf0, buf1, hist, outbuf, base_v, sem0, sem1):
    rows_per_w = x_hbm.shape[0] // NW
    nchunks = rows_per_w // CROWS
    wid = lax.axis_index("s") * NC + lax.axis_index("c")
    rowbase = wid * rows_per_w

    laneoff = lax.iota(jnp.int32, L) * jnp.int32(NBINS)
    ones = jnp.ones((L,), jnp.int32)

    _zero_hist(hist)
    pltpu.sync_copy(base_hbm, base_v)
    bvec = base_v[...]

    def _process(buf):
        @plsc.parallel_loop(0, CROWS * COLS, step=L, unroll=8)
        def _inner(i):
            key = lax.bitcast_convert_type(lax.abs(_load_vreg(buf, i)),
                                           jnp.int32)
            m = key >= bvec
            clamp = jnp.int32(WBINS) + lax.bitwise_and(
                lax.shift_right_logical(i, jnp.int32(4)), jnp.int32(7))
            binv = lax.min(
                lax.shift_right_logical(key - bvec, jnp.int32(7)), clamp)
            plsc.addupdate_scatter(hist, [laneoff + binv], ones, mask=m)

    _scan_loop(x_hbm, buf0, buf1, sem0, sem1, rowbase, nchunks, _process)
    _finish_hist(hist, outbuf, hist_hbm, wid)


def _sc_kernel(body):
    mesh = plsc.VectorSubcoreMesh(core_axis_name="c", subcore_axis_name="s",
                                  num_cores=NC, num_subcores=NS)
    return pl.kernel(
        body,
        out_type=jax.ShapeDtypeStruct((NW * NBINS,), jnp.int32),
        mesh=mesh,
        compiler_params=pltpu.CompilerParams(
            needs_layout_passes=False, use_tc_tiling_on_sc=True),
        scratch_types=[
            pltpu.VMEM((CROWS, COLS), jnp.float32),
            pltpu.VMEM((CROWS, COLS), jnp.float32),
            pltpu.VMEM((L * NBINS,), jnp.int32),
            pltpu.VMEM((NBINS,), jnp.int32),
            pltpu.VMEM((L,), jnp.int32),
            pltpu.SemaphoreType.DMA,
            pltpu.SemaphoreType.DMA,
        ],
    )


def _make_radix_kernel(shift_bin, shift_prefix, frac=1):
    return _sc_kernel(
        functools.partial(_radix_body, shift_bin, shift_prefix, frac))


def _select(hist, r):
    suffix = jnp.cumsum(hist[::-1])[::-1]
    b = jnp.sum(suffix >= r).astype(jnp.int32) - 1
    r_next = r - (suffix[b] - hist[b])
    return b, r_next


def _mask_body(kt_ref, x_ref, o_ref):
    bits = lax.bitcast_convert_type(x_ref[...], jnp.int32)
    key = jnp.bitwise_and(bits, jnp.int32(0x7FFFFFFF))
    o_ref[...] = jnp.where(key >= kt_ref[0], x_ref[...], jnp.float32(0.0))


def kernel(x):
    n = x.size
    k = max(1, int(n * (1.0 - 0.7)))
    rows = n // COLS
    x2d = x.reshape(rows, COLS)
    zeros16 = jnp.zeros((L,), jnp.int32)

    hs = _make_radix_kernel(19, 31, SAMPLE_FRAC)(x2d, zeros16)
    ks = max(1, k // SAMPLE_FRAC)
    bs, rs = _select(hs.reshape(NW, NBINS).sum(axis=0), jnp.int32(ks))
    cs = jnp.maximum(hs.reshape(NW, NBINS).sum(axis=0)[bs], 1)
    off = ((cs - rs).astype(jnp.float32) / cs.astype(jnp.float32)
           * jnp.float32(1 << 19)).astype(jnp.int32)
    t_est = (bs << 19) + jnp.clip(off, 0, (1 << 19) - 1)
    base = jnp.clip(t_est - (1 << 18), 0, 0x7FFFFFFF - (NBINS << 7))

    hw = _sc_kernel(_window_body)(x2d, jnp.full((L,), base, jnp.int32))
    suffix = jnp.cumsum(hw.reshape(NW, NBINS).sum(axis=0)[::-1])[::-1]
    j = jnp.sum(suffix >= k).astype(jnp.int32) - 1
    key_t_fast = base + (j << 7)
    missed = (j < 0) | (j >= WBINS)

    def _exact(_):
        h1 = _make_radix_kernel(19, 31)(x2d, zeros16)
        b1, r1 = _select(h1.reshape(NW, NBINS).sum(axis=0), jnp.int32(k))
        h2 = _make_radix_kernel(7, 19)(x2d, jnp.full((L,), b1, jnp.int32))
        b2, _ = _select(h2.reshape(NW, NBINS).sum(axis=0), r1)
        return ((b1 << 12) | b2) << 7

    key_t = lax.cond(missed, _exact, lambda _: key_t_fast, None)

    blk = 512
    out = pl.pallas_call(
        _mask_body,
        grid=(rows // blk,),
        in_specs=[
            pl.BlockSpec(memory_space=pltpu.SMEM),
            pl.BlockSpec((blk, COLS), lambda i: (i, 0)),
        ],
        out_specs=pl.BlockSpec((blk, COLS), lambda i: (i, 0)),
        out_shape=jax.ShapeDtypeStruct((rows, COLS), jnp.float32),
    )(key_t.reshape(1), x2d)
    return out.reshape(x.shape)


# speedup vs baseline: 1.4773x; 1.0077x over previous
import functools

import jax
import jax.numpy as jnp
from jax import lax
from jax.experimental import pallas as pl
from jax.experimental.pallas import tpu as pltpu
from jax.experimental.pallas import tpu_sc as plsc

NC = 2
NS = 16
L = 16
NW = NC * NS

NBINS = 4096
COLS = 2048
CROWS = 8
SAMPLE_FRAC = 16
WNBINS = 2048
WCROWS = 16
WUSE = WNBINS - 8


def _scan_loop(x_hbm, buf0, buf1, sem0, sem1, rowbase, nchunks, crows,
               process):
    pltpu.async_copy(x_hbm.at[pl.ds(rowbase, crows)], buf0, sem0)
    pltpu.async_copy(x_hbm.at[pl.ds(rowbase + crows, crows)], buf1, sem1)

    def _outer(g, c):
        row = rowbase + 2 * g * crows
        pltpu.make_async_copy(
            x_hbm.at[pl.ds(row, crows)], buf0, sem0).wait()
        process(buf0)

        @pl.when(2 * g + 2 < nchunks)
        def _():
            pltpu.async_copy(
                x_hbm.at[pl.ds(row + 2 * crows, crows)], buf0, sem0)

        pltpu.make_async_copy(
            x_hbm.at[pl.ds(row + crows, crows)], buf1, sem1).wait()
        process(buf1)

        @pl.when(2 * g + 3 < nchunks)
        def _():
            pltpu.async_copy(
                x_hbm.at[pl.ds(row + 3 * crows, crows)], buf1, sem1)
        return c
    lax.fori_loop(0, nchunks // 2, _outer, 0)


def _finish_hist(hist, outbuf, hist_hbm, wid, nbins):
    @plsc.parallel_loop(0, nbins, step=L)
    def _red(j):
        acc = hist[pl.ds(j, L)]
        for lane in range(1, L):
            acc = acc + hist[pl.ds(lane * nbins + j, L)]
        outbuf[pl.ds(j, L)] = acc

    pltpu.sync_copy(outbuf, hist_hbm.at[pl.ds(wid * nbins, nbins)])


def _zero_hist(hist, nbins):
    @plsc.parallel_loop(0, L * nbins, step=L, unroll=8)
    def _zero(i):
        hist[pl.ds(i, L)] = jnp.zeros((L,), jnp.int32)


def _load_vreg(buf, i):
    return buf[lax.shift_right_logical(i, COLS.bit_length() - 1),
               pl.ds(lax.bitwise_and(i, COLS - 1), L)]


def _radix_body(shift_bin, shift_prefix, frac, x_hbm, prefix_hbm, hist_hbm,
                buf0, buf1, hist, outbuf, prefix_v, sem0, sem1):
    rows_per_w = x_hbm.shape[0] // NW
    nchunks = rows_per_w // CROWS // frac
    wid = lax.axis_index("s") * NC + lax.axis_index("c")
    rowbase = wid * rows_per_w

    laneoff = lax.iota(jnp.int32, L) * jnp.int32(NBINS)
    ones = jnp.ones((L,), jnp.int32)

    _zero_hist(hist, NBINS)
    pltpu.sync_copy(prefix_hbm, prefix_v)
    pvec = prefix_v[...]

    def _process(buf):
        @plsc.parallel_loop(0, CROWS * COLS, step=L, unroll=8)
        def _inner(i):
            key = lax.bitcast_convert_type(lax.abs(_load_vreg(buf, i)),
                                           jnp.int32)
            t = lax.shift_right_logical(key, jnp.int32(shift_bin))
            idx = laneoff + lax.bitwise_and(t, jnp.int32(NBINS - 1))
            if shift_prefix >= 31:
                plsc.addupdate_scatter(hist, [idx], ones)
            else:
                m = lax.shift_right_logical(
                    t, jnp.int32(shift_prefix - shift_bin)) == pvec
                plsc.addupdate_scatter(hist, [idx], ones, mask=m)

    _scan_loop(x_hbm, buf0, buf1, sem0, sem1, rowbase, nchunks, CROWS,
               _process)
    _finish_hist(hist, outbuf, hist_hbm, wid, NBINS)


def _window_body(x_hbm, base_hbm, hist_hbm,
                 buf0, buf1, hist, outbuf, base_v, sem0, sem1):
    rows_per_w = x_hbm.shape[0] // NW
    nchunks = rows_per_w // WCROWS
    wid = lax.axis_index("s") * NC + lax.axis_index("c")
    rowbase = wid * rows_per_w

    laneoff = lax.iota(jnp.int32, L) * jnp.int32(WNBINS)
    ones = jnp.ones((L,), jnp.int32)

    _zero_hist(hist, WNBINS)
    pltpu.sync_copy(base_hbm, base_v)
    bvec = base_v[...]

    def _process(buf):
        @plsc.parallel_loop(0, WCROWS * COLS, step=L, unroll=8)
        def _inner(i):
            key = lax.bitcast_convert_type(lax.abs(_load_vreg(buf, i)),
                                           jnp.int32)
            m = key >= bvec
            clamp = jnp.int32(WUSE) + lax.bitwise_and(
                lax.shift_right_logical(i, jnp.int32(4)), jnp.int32(7))
            binv = lax.min(
                lax.shift_right_logical(key - bvec, jnp.int32(7)), clamp)
            plsc.addupdate_scatter(hist, [laneoff + binv], ones, mask=m)

    _scan_loop(x_hbm, buf0, buf1, sem0, sem1, rowbase, nchunks, WCROWS,
               _process)
    _finish_hist(hist, outbuf, hist_hbm, wid, WNBINS)


def _sc_kernel(body, nbins, crows):
    mesh = plsc.VectorSubcoreMesh(core_axis_name="c", subcore_axis_name="s",
                                  num_cores=NC, num_subcores=NS)
    return pl.kernel(
        body,
        out_type=jax.ShapeDtypeStruct((NW * nbins,), jnp.int32),
        mesh=mesh,
        compiler_params=pltpu.CompilerParams(
            needs_layout_passes=False, use_tc_tiling_on_sc=True),
        scratch_types=[
            pltpu.VMEM((crows, COLS), jnp.float32),
            pltpu.VMEM((crows, COLS), jnp.float32),
            pltpu.VMEM((L * nbins,), jnp.int32),
            pltpu.VMEM((nbins,), jnp.int32),
            pltpu.VMEM((L,), jnp.int32),
            pltpu.SemaphoreType.DMA,
            pltpu.SemaphoreType.DMA,
        ],
    )


def _make_radix_kernel(shift_bin, shift_prefix, frac=1):
    return _sc_kernel(
        functools.partial(_radix_body, shift_bin, shift_prefix, frac),
        NBINS, CROWS)


def _select(hist, r):
    suffix = jnp.cumsum(hist[::-1])[::-1]
    b = jnp.sum(suffix >= r).astype(jnp.int32) - 1
    r_next = r - (suffix[b] - hist[b])
    return b, r_next


def _mask_body(kt_ref, x_ref, o_ref):
    bits = lax.bitcast_convert_type(x_ref[...], jnp.int32)
    key = jnp.bitwise_and(bits, jnp.int32(0x7FFFFFFF))
    o_ref[...] = jnp.where(key >= kt_ref[0], x_ref[...], jnp.float32(0.0))


def kernel(x):
    n = x.size
    k = max(1, int(n * (1.0 - 0.7)))
    rows = n // COLS
    x2d = x.reshape(rows, COLS)
    zeros16 = jnp.zeros((L,), jnp.int32)

    hs = _make_radix_kernel(19, 31, SAMPLE_FRAC)(x2d, zeros16)
    ks = max(1, k // SAMPLE_FRAC)
    bs, rs = _select(hs.reshape(NW, NBINS).sum(axis=0), jnp.int32(ks))
    cs = jnp.maximum(hs.reshape(NW, NBINS).sum(axis=0)[bs], 1)
    off = ((cs - rs).astype(jnp.float32) / cs.astype(jnp.float32)
           * jnp.float32(1 << 19)).astype(jnp.int32)
    t_est = (bs << 19) + jnp.clip(off, 0, (1 << 19) - 1)
    base = jnp.clip(t_est - (WNBINS << 6), 0, 0x7FFFFFFF - (WNBINS << 7))

    hw = _sc_kernel(_window_body, WNBINS, WCROWS)(
        x2d, jnp.full((L,), base, jnp.int32))
    suffix = jnp.cumsum(hw.reshape(NW, WNBINS).sum(axis=0)[::-1])[::-1]
    j = jnp.sum(suffix >= k).astype(jnp.int32) - 1
    key_t_fast = base + (j << 7)
    missed = (j < 0) | (j >= WUSE)

    def _exact(_):
        h1 = _make_radix_kernel(19, 31)(x2d, zeros16)
        b1, r1 = _select(h1.reshape(NW, NBINS).sum(axis=0), jnp.int32(k))
        h2 = _make_radix_kernel(7, 19)(x2d, jnp.full((L,), b1, jnp.int32))
        b2, _ = _select(h2.reshape(NW, NBINS).sum(axis=0), r1)
        return ((b1 << 12) | b2) << 7

    key_t = lax.cond(missed, _exact, lambda _: key_t_fast, None)

    blk = 1024
    out = pl.pallas_call(
        _mask_body,
        grid=(rows // blk,),
        in_specs=[
            pl.BlockSpec(memory_space=pltpu.SMEM),
            pl.BlockSpec((blk, COLS), lambda i: (i, 0)),
        ],
        out_specs=pl.BlockSpec((blk, COLS), lambda i: (i, 0)),
        out_shape=jax.ShapeDtypeStruct((rows, COLS), jnp.float32),
    )(key_t.reshape(1), x2d)
    return out.reshape(x.shape)


# speedup vs baseline: 1.5552x; 1.0527x over previous
import functools

import jax
import jax.numpy as jnp
from jax import lax
from jax.experimental import pallas as pl
from jax.experimental.pallas import tpu as pltpu
from jax.experimental.pallas import tpu_sc as plsc

NC = 2
NS = 16
L = 16
NW = NC * NS

NBINS = 4096
COLS = 2048
CROWS = 8
SAMPLE_FRAC = 32
WNBINS = 2048
WCROWS = 16
WUSE = WNBINS - 16


def _scan_loop(x_hbm, buf0, buf1, sem0, sem1, rowbase, nchunks, crows,
               process):
    pltpu.async_copy(x_hbm.at[pl.ds(rowbase, crows)], buf0, sem0)
    pltpu.async_copy(x_hbm.at[pl.ds(rowbase + crows, crows)], buf1, sem1)

    def _outer(g, c):
        row = rowbase + 2 * g * crows
        pltpu.make_async_copy(
            x_hbm.at[pl.ds(row, crows)], buf0, sem0).wait()
        process(buf0)

        @pl.when(2 * g + 2 < nchunks)
        def _():
            pltpu.async_copy(
                x_hbm.at[pl.ds(row + 2 * crows, crows)], buf0, sem0)

        pltpu.make_async_copy(
            x_hbm.at[pl.ds(row + crows, crows)], buf1, sem1).wait()
        process(buf1)

        @pl.when(2 * g + 3 < nchunks)
        def _():
            pltpu.async_copy(
                x_hbm.at[pl.ds(row + 3 * crows, crows)], buf1, sem1)
        return c
    lax.fori_loop(0, nchunks // 2, _outer, 0)


def _finish_hist(hist, outbuf, hist_hbm, wid, nbins):
    @plsc.parallel_loop(0, nbins, step=L)
    def _red(j):
        acc = hist[pl.ds(j, L)]
        for lane in range(1, L):
            acc = acc + hist[pl.ds(lane * nbins + j, L)]
        outbuf[pl.ds(j, L)] = acc

    pltpu.sync_copy(outbuf, hist_hbm.at[pl.ds(wid * nbins, nbins)])


def _zero_hist(hist, nbins):
    @plsc.parallel_loop(0, L * nbins, step=L, unroll=8)
    def _zero(i):
        hist[pl.ds(i, L)] = jnp.zeros((L,), jnp.int32)


def _load_vreg(buf, i):
    return buf[lax.shift_right_logical(i, COLS.bit_length() - 1),
               pl.ds(lax.bitwise_and(i, COLS - 1), L)]


def _radix_body(shift_bin, shift_prefix, frac, x_hbm, prefix_hbm, hist_hbm,
                buf0, buf1, hist, outbuf, prefix_v, sem0, sem1):
    rows_per_w = x_hbm.shape[0] // NW
    nchunks = rows_per_w // CROWS // frac
    wid = lax.axis_index("s") * NC + lax.axis_index("c")
    rowbase = wid * rows_per_w

    laneoff = lax.iota(jnp.int32, L) * jnp.int32(NBINS)
    ones = jnp.ones((L,), jnp.int32)

    _zero_hist(hist, NBINS)
    pltpu.sync_copy(prefix_hbm, prefix_v)
    pvec = prefix_v[...]

    def _process(buf):
        @plsc.parallel_loop(0, CROWS * COLS, step=L, unroll=8)
        def _inner(i):
            key = lax.bitcast_convert_type(lax.abs(_load_vreg(buf, i)),
                                           jnp.int32)
            t = lax.shift_right_logical(key, jnp.int32(shift_bin))
            idx = laneoff + lax.bitwise_and(t, jnp.int32(NBINS - 1))
            if shift_prefix >= 31:
                plsc.addupdate_scatter(hist, [idx], ones)
            else:
                m = lax.shift_right_logical(
                    t, jnp.int32(shift_prefix - shift_bin)) == pvec
                plsc.addupdate_scatter(hist, [idx], ones, mask=m)

    _scan_loop(x_hbm, buf0, buf1, sem0, sem1, rowbase, nchunks, CROWS,
               _process)
    _finish_hist(hist, outbuf, hist_hbm, wid, NBINS)


def _window_body(x_hbm, base_hbm, hist_hbm,
                 buf0, buf1, hist, outbuf, base_v, sem0, sem1):
    rows_per_w = x_hbm.shape[0] // NW
    nchunks = rows_per_w // WCROWS
    wid = lax.axis_index("s") * NC + lax.axis_index("c")
    rowbase = wid * rows_per_w

    laneoff = lax.iota(jnp.int32, L) * jnp.int32(WNBINS)
    ones = jnp.ones((L,), jnp.int32)

    _zero_hist(hist, WNBINS)
    pltpu.sync_copy(base_hbm, base_v)
    bvec = base_v[...]

    def _process(buf):
        @plsc.parallel_loop(0, WCROWS * COLS, step=L, unroll=16)
        def _inner(i):
            key = lax.bitcast_convert_type(lax.abs(_load_vreg(buf, i)),
                                           jnp.int32)
            m = key >= bvec
            clamp = jnp.int32(WUSE) + lax.bitwise_and(
                lax.shift_right_logical(i, jnp.int32(4)), jnp.int32(15))
            binv = lax.min(
                lax.shift_right_logical(key - bvec, jnp.int32(7)), clamp)
            plsc.addupdate_scatter(hist, [laneoff + binv], ones, mask=m)

    _scan_loop(x_hbm, buf0, buf1, sem0, sem1, rowbase, nchunks, WCROWS,
               _process)
    _finish_hist(hist, outbuf, hist_hbm, wid, WNBINS)


def _sc_kernel(body, nbins, crows):
    mesh = plsc.VectorSubcoreMesh(core_axis_name="c", subcore_axis_name="s",
                                  num_cores=NC, num_subcores=NS)
    return pl.kernel(
        body,
        out_type=jax.ShapeDtypeStruct((NW * nbins,), jnp.int32),
        mesh=mesh,
        compiler_params=pltpu.CompilerParams(
            needs_layout_passes=False, use_tc_tiling_on_sc=True),
        scratch_types=[
            pltpu.VMEM((crows, COLS), jnp.float32),
            pltpu.VMEM((crows, COLS), jnp.float32),
            pltpu.VMEM((L * nbins,), jnp.int32),
            pltpu.VMEM((nbins,), jnp.int32),
            pltpu.VMEM((L,), jnp.int32),
            pltpu.SemaphoreType.DMA,
            pltpu.SemaphoreType.DMA,
        ],
    )


def _make_radix_kernel(shift_bin, shift_prefix, frac=1):
    return _sc_kernel(
        functools.partial(_radix_body, shift_bin, shift_prefix, frac),
        NBINS, CROWS)


def _select(hist, r):
    suffix = jnp.cumsum(hist[::-1])[::-1]
    b = jnp.sum(suffix >= r).astype(jnp.int32) - 1
    r_next = r - (suffix[b] - hist[b])
    return b, r_next


def _mask_body(kt_ref, x_ref, o_ref):
    bits = lax.bitcast_convert_type(x_ref[...], jnp.int32)
    key = jnp.bitwise_and(bits, jnp.int32(0x7FFFFFFF))
    o_ref[...] = jnp.where(key >= kt_ref[0], x_ref[...], jnp.float32(0.0))


def kernel(x):
    n = x.size
    k = max(1, int(n * (1.0 - 0.7)))
    rows = n // COLS
    x2d = x.reshape(rows, COLS)
    zeros16 = jnp.zeros((L,), jnp.int32)

    hs = _make_radix_kernel(19, 31, SAMPLE_FRAC)(x2d, zeros16)
    ks = max(1, k // SAMPLE_FRAC)
    bs, rs = _select(hs.reshape(NW, NBINS).sum(axis=0), jnp.int32(ks))
    cs = jnp.maximum(hs.reshape(NW, NBINS).sum(axis=0)[bs], 1)
    off = ((cs - rs).astype(jnp.float32) / cs.astype(jnp.float32)
           * jnp.float32(1 << 19)).astype(jnp.int32)
    t_est = (bs << 19) + jnp.clip(off, 0, (1 << 19) - 1)
    base = jnp.clip(t_est - (WNBINS << 6), 0, 0x7FFFFFFF - (WNBINS << 7))

    hw = _sc_kernel(_window_body, WNBINS, WCROWS)(
        x2d, jnp.full((L,), base, jnp.int32))
    suffix = jnp.cumsum(hw.reshape(NW, WNBINS).sum(axis=0)[::-1])[::-1]
    j = jnp.sum(suffix >= k).astype(jnp.int32) - 1
    key_t_fast = base + (j << 7)
    missed = (j < 0) | (j >= WUSE)

    def _exact(_):
        h1 = _make_radix_kernel(19, 31)(x2d, zeros16)
        b1, r1 = _select(h1.reshape(NW, NBINS).sum(axis=0), jnp.int32(k))
        h2 = _make_radix_kernel(7, 19)(x2d, jnp.full((L,), b1, jnp.int32))
        b2, _ = _select(h2.reshape(NW, NBINS).sum(axis=0), r1)
        return ((b1 << 12) | b2) << 7

    key_t = lax.cond(missed, _exact, lambda _: key_t_fast, None)

    blk = 1024
    out = pl.pallas_call(
        _mask_body,
        grid=(rows // blk,),
        in_specs=[
            pl.BlockSpec(memory_space=pltpu.SMEM),
            pl.BlockSpec((blk, COLS), lambda i: (i, 0)),
        ],
        out_specs=pl.BlockSpec((blk, COLS), lambda i: (i, 0)),
        out_shape=jax.ShapeDtypeStruct((rows, COLS), jnp.float32),
    )(key_t.reshape(1), x2d)
    return out.reshape(x.shape)


# speedup vs baseline: 1.5729x; 1.0114x over previous
import functools

import jax
import jax.numpy as jnp
from jax import lax
from jax.experimental import pallas as pl
from jax.experimental.pallas import tpu as pltpu
from jax.experimental.pallas import tpu_sc as plsc

NC = 2
NS = 16
L = 16
NW = NC * NS

NBINS = 4096
COLS = 2048
CROWS = 8
SAMPLE_FRAC = 32
WNBINS = 2048
WCROWS = 16
WUSE = WNBINS - 16


def _scan_loop(x_hbm, buf0, buf1, sem0, sem1, rowbase, nchunks, crows,
               process):
    pltpu.async_copy(x_hbm.at[pl.ds(rowbase, crows)], buf0, sem0)
    pltpu.async_copy(x_hbm.at[pl.ds(rowbase + crows, crows)], buf1, sem1)

    def _outer(g, c):
        row = rowbase + 2 * g * crows
        pltpu.make_async_copy(
            x_hbm.at[pl.ds(row, crows)], buf0, sem0).wait()
        process(buf0)

        @pl.when(2 * g + 2 < nchunks)
        def _():
            pltpu.async_copy(
                x_hbm.at[pl.ds(row + 2 * crows, crows)], buf0, sem0)

        pltpu.make_async_copy(
            x_hbm.at[pl.ds(row + crows, crows)], buf1, sem1).wait()
        process(buf1)

        @pl.when(2 * g + 3 < nchunks)
        def _():
            pltpu.async_copy(
                x_hbm.at[pl.ds(row + 3 * crows, crows)], buf1, sem1)
        return c
    lax.fori_loop(0, nchunks // 2, _outer, 0)


def _finish_hist(hist, outbuf, hist_hbm, wid, nbins):
    @plsc.parallel_loop(0, nbins, step=L)
    def _red(j):
        acc = hist[pl.ds(j, L)]
        for lane in range(1, L):
            acc = acc + hist[pl.ds(lane * nbins + j, L)]
        outbuf[pl.ds(j, L)] = acc

    pltpu.sync_copy(outbuf, hist_hbm.at[pl.ds(wid * nbins, nbins)])


def _zero_hist(hist, nbins):
    @plsc.parallel_loop(0, L * nbins, step=L, unroll=8)
    def _zero(i):
        hist[pl.ds(i, L)] = jnp.zeros((L,), jnp.int32)


def _load_vreg(buf, i):
    return buf[lax.shift_right_logical(i, COLS.bit_length() - 1),
               pl.ds(lax.bitwise_and(i, COLS - 1), L)]


def _radix_body(shift_bin, shift_prefix, frac, nbins, x_hbm, prefix_hbm,
                hist_hbm, buf0, buf1, hist, outbuf, prefix_v, sem0, sem1):
    rows_per_w = x_hbm.shape[0] // NW
    nchunks = rows_per_w // CROWS // frac
    wid = lax.axis_index("s") * NC + lax.axis_index("c")
    rowbase = wid * rows_per_w

    laneoff = lax.iota(jnp.int32, L) * jnp.int32(nbins)
    ones = jnp.ones((L,), jnp.int32)

    _zero_hist(hist, nbins)
    pltpu.sync_copy(prefix_hbm, prefix_v)
    pvec = prefix_v[...]

    def _process(buf):
        @plsc.parallel_loop(0, CROWS * COLS, step=L, unroll=8)
        def _inner(i):
            key = lax.bitcast_convert_type(lax.abs(_load_vreg(buf, i)),
                                           jnp.int32)
            t = lax.shift_right_logical(key, jnp.int32(shift_bin))
            idx = laneoff + lax.bitwise_and(t, jnp.int32(nbins - 1))
            if shift_prefix >= 31:
                plsc.addupdate_scatter(hist, [idx], ones)
            else:
                m = lax.shift_right_logical(
                    t, jnp.int32(shift_prefix - shift_bin)) == pvec
                plsc.addupdate_scatter(hist, [idx], ones, mask=m)

    _scan_loop(x_hbm, buf0, buf1, sem0, sem1, rowbase, nchunks, CROWS,
               _process)
    _finish_hist(hist, outbuf, hist_hbm, wid, nbins)


def _window_body(x_hbm, base_hbm, hist_hbm,
                 buf0, buf1, hist, outbuf, base_v, sem0, sem1):
    rows_per_w = x_hbm.shape[0] // NW
    nchunks = rows_per_w // WCROWS
    wid = lax.axis_index("s") * NC + lax.axis_index("c")
    rowbase = wid * rows_per_w

    laneoff = lax.iota(jnp.int32, L) * jnp.int32(WNBINS)
    ones = jnp.ones((L,), jnp.int32)

    _zero_hist(hist, WNBINS)
    pltpu.sync_copy(base_hbm, base_v)
    bvec = base_v[...]

    def _process(buf):
        @plsc.parallel_loop(0, WCROWS * COLS, step=L, unroll=16)
        def _inner(i):
            key = lax.bitcast_convert_type(lax.abs(_load_vreg(buf, i)),
                                           jnp.int32)
            m = key >= bvec
            clamp = jnp.int32(WUSE) + lax.bitwise_and(
                lax.shift_right_logical(i, jnp.int32(4)), jnp.int32(15))
            binv = lax.min(
                lax.shift_right_logical(key - bvec, jnp.int32(7)), clamp)
            plsc.addupdate_scatter(hist, [laneoff + binv], ones, mask=m)

    _scan_loop(x_hbm, buf0, buf1, sem0, sem1, rowbase, nchunks, WCROWS,
               _process)
    _finish_hist(hist, outbuf, hist_hbm, wid, WNBINS)


def _sc_kernel(body, nbins, crows):
    mesh = plsc.VectorSubcoreMesh(core_axis_name="c", subcore_axis_name="s",
                                  num_cores=NC, num_subcores=NS)
    return pl.kernel(
        body,
        out_type=jax.ShapeDtypeStruct((NW * nbins,), jnp.int32),
        mesh=mesh,
        compiler_params=pltpu.CompilerParams(
            needs_layout_passes=False, use_tc_tiling_on_sc=True),
        scratch_types=[
            pltpu.VMEM((crows, COLS), jnp.float32),
            pltpu.VMEM((crows, COLS), jnp.float32),
            pltpu.VMEM((L * nbins,), jnp.int32),
            pltpu.VMEM((nbins,), jnp.int32),
            pltpu.VMEM((L,), jnp.int32),
            pltpu.SemaphoreType.DMA,
            pltpu.SemaphoreType.DMA,
        ],
    )


def _make_radix_kernel(shift_bin, shift_prefix, frac=1, nbins=NBINS):
    return _sc_kernel(
        functools.partial(_radix_body, shift_bin, shift_prefix, frac, nbins),
        nbins, CROWS)


def _select(hist, r):
    suffix = jnp.cumsum(hist[::-1])[::-1]
    b = jnp.sum(suffix >= r).astype(jnp.int32) - 1
    r_next = r - (suffix[b] - hist[b])
    return b, r_next


def _mask_body(kt_ref, x_ref, o_ref):
    bits = lax.bitcast_convert_type(x_ref[...], jnp.int32)
    key = jnp.bitwise_and(bits, jnp.int32(0x7FFFFFFF))
    o_ref[...] = jnp.where(key >= kt_ref[0], x_ref[...], jnp.float32(0.0))


def kernel(x):
    n = x.size
    k = max(1, int(n * (1.0 - 0.7)))
    rows = n // COLS
    x2d = x.reshape(rows, COLS)
    zeros16 = jnp.zeros((L,), jnp.int32)

    hs = _make_radix_kernel(20, 31, SAMPLE_FRAC, WNBINS)(x2d, zeros16)
    ks = max(1, k // SAMPLE_FRAC)
    hsum = hs.reshape(NW, WNBINS).sum(axis=0)
    bs, rs = _select(hsum, jnp.int32(ks))
    cs = jnp.maximum(hsum[bs], 1)
    off = ((cs - rs).astype(jnp.float32) / cs.astype(jnp.float32)
           * jnp.float32(1 << 20)).astype(jnp.int32)
    t_est = (bs << 20) + jnp.clip(off, 0, (1 << 20) - 1)
    base = jnp.clip(t_est - (WNBINS << 6), 0, 0x7FFFFFFF - (WNBINS << 7))

    hw = _sc_kernel(_window_body, WNBINS, WCROWS)(
        x2d, jnp.full((L,), base, jnp.int32))
    suffix = jnp.cumsum(hw.reshape(NW, WNBINS).sum(axis=0)[::-1])[::-1]
    j = jnp.sum(suffix >= k).astype(jnp.int32) - 1
    key_t_fast = base + (j << 7)
    missed = (j < 0) | (j >= WUSE)

    def _exact(_):
        h1 = _make_radix_kernel(19, 31)(x2d, zeros16)
        b1, r1 = _select(h1.reshape(NW, NBINS).sum(axis=0), jnp.int32(k))
        h2 = _make_radix_kernel(7, 19)(x2d, jnp.full((L,), b1, jnp.int32))
        b2, _ = _select(h2.reshape(NW, NBINS).sum(axis=0), r1)
        return ((b1 << 12) | b2) << 7

    key_t = lax.cond(missed, _exact, lambda _: key_t_fast, None)

    blk = 1024
    out = pl.pallas_call(
        _mask_body,
        grid=(rows // blk,),
        in_specs=[
            pl.BlockSpec(memory_space=pltpu.SMEM),
            pl.BlockSpec((blk, COLS), lambda i: (i, 0)),
        ],
        out_specs=pl.BlockSpec((blk, COLS), lambda i: (i, 0)),
        out_shape=jax.ShapeDtypeStruct((rows, COLS), jnp.float32),
    )(key_t.reshape(1), x2d)
    return out.reshape(x.shape)
